# baseline (speedup 1.0000x reference)
import jax
import jax.numpy as jnp
from jax import lax
from jax.experimental import pallas as pl
from jax.experimental.pallas import tpu as pltpu
from jax.experimental.pallas import tpu_sc as plsc

B, L, V, D = 4096, 50, 100000, 300
DA = 256
TAIL0 = D - 128
DT = 48
NC, NS = 2, 16
NW = NC * NS
BPW = B // NW
LANES = 16


def _pool_a_body(batcht_hbm, piece_hbm, pooled_hbm, idx_s, rows0, rows1,
                 out_v, sem0, sem1):
    nch = DA // LANES
    wid = lax.axis_index("s") * NC + lax.axis_index("c")
    base = wid * BPW
    pltpu.sync_copy(batcht_hbm.at[:, pl.ds(base, BPW)], idx_s)

    bufs = ((rows0, sem0), (rows1, sem1))

    def gather_copies(r, buf, sem):
        return [
            pltpu.make_async_copy(
                piece_hbm.at[idx_s.at[r], pl.ds(ct * 128, 128)],
                buf.at[:, pl.ds(ct * 128, 128)], sem)
            for ct in range(DA // 128)
        ]

    def gather_start(r, buf, sem):
        for c in gather_copies(r, buf, sem):
            c.start()

    def gather_wait(r, buf, sem):
        for c in gather_copies(r, buf, sem):
            c.wait()

    gather_start(0, rows0, sem0)
    gather_start(1, rows1, sem1)

    zero = jnp.zeros((LANES,), jnp.float32)

    def zbody(e, carry):
        for j in range(nch):
            out_v[e, pl.ds(LANES * j, LANES)] = zero
        return carry

    lax.fori_loop(0, BPW, zbody, 0, unroll=2)

    def accumulate(buf):
        def ebody(e, carry):
            for j in range(nch):
                plsc.addupdate(out_v.at[e, pl.ds(LANES * j, LANES)],
                               buf[e, pl.ds(LANES * j, LANES)])
            return carry
        lax.fori_loop(0, BPW, ebody, 0, unroll=2)

    def pair(i, carry):
        r0 = i * 2
        for b in range(2):
            buf, sem = bufs[b]
            r = r0 + b
            gather_wait(r, buf, sem)
            accumulate(buf)
            nxt = r + 2

            @pl.when(nxt < L)
            def _():
                gather_start(nxt, buf, sem)
        return carry

    lax.fori_loop(0, L // 2, pair, 0)
    pltpu.sync_copy(out_v, pooled_hbm.at[pl.ds(base, BPW)])


def _pool_b_body(batcht_hbm, tail_hbm, pooled_hbm, idx_s, rows0, rows1,
                 out_v, sem0, sem1):
    offs = (80, 96, 112)
    wid = lax.axis_index("s") * NC + lax.axis_index("c")
    base = wid * BPW
    pltpu.sync_copy(batcht_hbm.at[:, pl.ds(base, BPW)], idx_s)

    bufs = ((rows0, sem0), (rows1, sem1))

    def gather(r, buf, sem):
        return pltpu.make_async_copy(tail_hbm.at[idx_s.at[r]], buf, sem)

    gather(0, rows0, sem0).start()
    gather(1, rows1, sem1).start()

    zero = jnp.zeros((LANES,), jnp.float32)

    def zbody(e, carry):
        for j in range(3):
            out_v[e, pl.ds(LANES * j, LANES)] = zero
        return carry

    lax.fori_loop(0, BPW, zbody, 0, unroll=2)

    def accumulate(buf):
        def ebody(e, carry):
            for j in range(3):
                plsc.addupdate(out_v.at[e, pl.ds(LANES * j, LANES)],
                               buf[e, pl.ds(offs[j], LANES)])
            return carry
        lax.fori_loop(0, BPW, ebody, 0, unroll=4)

    def pair(i, carry):
        r0 = i * 2
        for b in range(2):
            buf, sem = bufs[b]
            r = r0 + b
            gather(r, buf, sem).wait()
            accumulate(buf)
            nxt = r + 2

            @pl.when(nxt < L)
            def _():
                gather(nxt, buf, sem).start()
        return carry

    lax.fori_loop(0, L // 2, pair, 0)
    pltpu.sync_copy(out_v, pooled_hbm.at[pl.ds(base, BPW)])


def _make_pool(body, width, out_w):
    mesh = plsc.VectorSubcoreMesh(core_axis_name="c", subcore_axis_name="s")
    return pl.kernel(
        body,
        mesh=mesh,
        out_type=jax.ShapeDtypeStruct((B, out_w), jnp.float32),
        scratch_types=[
            pltpu.VMEM((L, BPW), jnp.int32),
            pltpu.VMEM((BPW, width), jnp.float32),
            pltpu.VMEM((BPW, width), jnp.float32),
            pltpu.VMEM((BPW, out_w), jnp.float32),
            pltpu.SemaphoreType.DMA,
            pltpu.SemaphoreType.DMA,
        ],
    )


def _mlp_body(xa_ref, xb_ref, lens_ref, w1a_ref, w1b_ref, b1_ref, w2_ref,
              b2_ref, w3_ref, b3_ref, o_ref):
    recip = 1.0 / lens_ref[...].astype(jnp.float32)
    xa = xa_ref[...] * recip
    xb = xb_ref[...] * recip
    cdims = (((1,), (1,)), ((), ()))
    h1 = (lax.dot_general(xa, w1a_ref[...], cdims,
                          preferred_element_type=jnp.float32)
          + lax.dot_general(xb, w1b_ref[...], cdims,
                            preferred_element_type=jnp.float32))
    h1 = jnp.maximum(h1 + b1_ref[...], 0.0)
    h2 = lax.dot_general(h1, w2_ref[...], cdims,
                         preferred_element_type=jnp.float32)
    h2 = jnp.maximum(h2 + b2_ref[...], 0.0)
    h3 = jnp.sum(h2 * w3_ref[...], axis=1, keepdims=True)
    o_ref[...] = h3 + b3_ref[0, 0]


def _mlp(pooled_a, pooled_b, lens, W1a, W1b, b1, W2, b2, W3, b3):
    BB = 512
    grid = (B // BB,)
    return pl.pallas_call(
        _mlp_body,
        grid=grid,
        in_specs=[
            pl.BlockSpec((BB, DA), lambda i: (i, 0)),
            pl.BlockSpec((BB, DT), lambda i: (i, 0)),
            pl.BlockSpec((BB, 1), lambda i: (i, 0)),
            pl.BlockSpec((150, DA), lambda i: (0, 0)),
            pl.BlockSpec((150, DT), lambda i: (0, 0)),
            pl.BlockSpec((1, 150), lambda i: (0, 0)),
            pl.BlockSpec((150, 150), lambda i: (0, 0)),
            pl.BlockSpec((1, 150), lambda i: (0, 0)),
            pl.BlockSpec((1, 150), lambda i: (0, 0)),
            pl.BlockSpec(memory_space=pltpu.MemorySpace.SMEM),
        ],
        out_specs=pl.BlockSpec((BB, 1), lambda i: (i, 0)),
        out_shape=jax.ShapeDtypeStruct((B, 1), jnp.float32),
    )(pooled_a, pooled_b, lens, W1a, W1b, b1, W2, b2, W3, b3)


def kernel(batch, lens, table, W1, b1, W2, b2, W3, b3):
    batcht = batch.T
    piece_a = table[:, :DA]
    tail_table = table[:, TAIL0:]
    pooled_a = _make_pool(_pool_a_body, DA, DA)(batcht, piece_a)
    tail2, pooled_a = lax.optimization_barrier((tail_table, pooled_a))
    pooled_b = _make_pool(_pool_b_body, 128, DT)(batcht, tail2)
    W1b = jnp.pad(W1[:, DA:], ((0, 0), (4, 0)))
    W1a = W1[:, :DA]
    lens2 = lens.reshape(B, 1)
    out = _mlp(pooled_a, pooled_b, lens2, W1a, W1b, b1.reshape(1, 150), W2,
               b2.reshape(1, 150), W3, b3.reshape(1, 1))
    return out.reshape((B,))


# speedup vs baseline: 1.1903x; 1.1903x over previous
import jax
import jax.numpy as jnp
from jax import lax
from jax.experimental import pallas as pl
from jax.experimental.pallas import tpu as pltpu
from jax.experimental.pallas import tpu_sc as plsc

B, L, V, D = 4096, 50, 100000, 300
DA = 256
TAIL0 = D - 128
DT = 48
NC, NS = 2, 16
NW = NC * NS
BPW = B // NW
LANES = 16


def _pool_a_body(batcht_hbm, piece_hbm, pooled_hbm, idx_s, rows0, rows1,
                 out_v, sem0, sem1):
    nch = DA // LANES
    wid = lax.axis_index("s") * NC + lax.axis_index("c")
    base = wid * BPW
    pltpu.sync_copy(batcht_hbm.at[:, pl.ds(base, BPW)], idx_s)

    bufs = ((rows0, sem0), (rows1, sem1))

    def gather_copies(r, buf, sem):
        return [
            pltpu.make_async_copy(
                piece_hbm.at[idx_s.at[r], pl.ds(ct * 128, 128)],
                buf.at[:, pl.ds(ct * 128, 128)], sem)
            for ct in range(DA // 128)
        ]

    def gather_start(r, buf, sem):
        for c in gather_copies(r, buf, sem):
            c.start()

    def gather_wait(r, buf, sem):
        for c in gather_copies(r, buf, sem):
            c.wait()

    gather_start(0, rows0, sem0)
    gather_start(1, rows1, sem1)

    zero = jnp.zeros((LANES,), jnp.float32)

    def zbody(e, carry):
        for j in range(nch):
            out_v[e, pl.ds(LANES * j, LANES)] = zero
        return carry

    lax.fori_loop(0, BPW, zbody, 0)

    def accumulate(buf):
        def ebody(e, carry):
            for j in range(nch):
                plsc.addupdate(out_v.at[e, pl.ds(LANES * j, LANES)],
                               buf[e, pl.ds(LANES * j, LANES)])
            return carry
        lax.fori_loop(0, BPW, ebody, 0)

    def pair(i, carry):
        r0 = i * 2
        for b in range(2):
            buf, sem = bufs[b]
            r = r0 + b
            gather_wait(r, buf, sem)
            accumulate(buf)
            nxt = r + 2

            @pl.when(nxt < L)
            def _():
                gather_start(nxt, buf, sem)
        return carry

    lax.fori_loop(0, L // 2, pair, 0)
    pltpu.sync_copy(out_v, pooled_hbm.at[pl.ds(base, BPW)])


def _pool_b_body(batcht_hbm, tail_hbm, pooled_hbm, idx_s, rows0, rows1,
                 out_v, sem0, sem1):
    offs = (80, 96, 112)
    wid = lax.axis_index("s") * NC + lax.axis_index("c")
    base = wid * BPW
    pltpu.sync_copy(batcht_hbm.at[:, pl.ds(base, BPW)], idx_s)

    bufs = ((rows0, sem0), (rows1, sem1))

    def gather(r, buf, sem):
        return pltpu.make_async_copy(tail_hbm.at[idx_s.at[r]], buf, sem)

    gather(0, rows0, sem0).start()
    gather(1, rows1, sem1).start()

    zero = jnp.zeros((LANES,), jnp.float32)

    def zbody(e, carry):
        for j in range(3):
            out_v[e, pl.ds(LANES * j, LANES)] = zero
        return carry

    lax.fori_loop(0, BPW, zbody, 0)

    def accumulate(buf):
        def ebody(e, carry):
            for j in range(3):
                plsc.addupdate(out_v.at[e, pl.ds(LANES * j, LANES)],
                               buf[e, pl.ds(offs[j], LANES)])
            return carry
        lax.fori_loop(0, BPW, ebody, 0)

    def pair(i, carry):
        r0 = i * 2
        for b in range(2):
            buf, sem = bufs[b]
            r = r0 + b
            gather(r, buf, sem).wait()
            accumulate(buf)
            nxt = r + 2

            @pl.when(nxt < L)
            def _():
                gather(nxt, buf, sem).start()
        return carry

    lax.fori_loop(0, L // 2, pair, 0)
    pltpu.sync_copy(out_v, pooled_hbm.at[pl.ds(base, BPW)])


def _make_pool(body, width, out_w):
    mesh = plsc.VectorSubcoreMesh(core_axis_name="c", subcore_axis_name="s")
    return pl.kernel(
        body,
        mesh=mesh,
        out_type=jax.ShapeDtypeStruct((B, out_w), jnp.float32),
        scratch_types=[
            pltpu.VMEM((L, BPW), jnp.int32),
            pltpu.VMEM((BPW, width), jnp.float32),
            pltpu.VMEM((BPW, width), jnp.float32),
            pltpu.VMEM((BPW, out_w), jnp.float32),
            pltpu.SemaphoreType.DMA,
            pltpu.SemaphoreType.DMA,
        ],
    )


def _mlp_body(xa_ref, xb_ref, lens_ref, w1a_ref, w1b_ref, b1_ref, w2_ref,
              b2_ref, w3_ref, b3_ref, o_ref):
    recip = 1.0 / lens_ref[...].astype(jnp.float32)
    xa = xa_ref[...] * recip
    xb = xb_ref[...] * recip
    cdims = (((1,), (1,)), ((), ()))
    h1 = (lax.dot_general(xa, w1a_ref[...], cdims,
                          preferred_element_type=jnp.float32)
          + lax.dot_general(xb, w1b_ref[...], cdims,
                            preferred_element_type=jnp.float32))
    h1 = jnp.maximum(h1 + b1_ref[...], 0.0)
    h2 = lax.dot_general(h1, w2_ref[...], cdims,
                         preferred_element_type=jnp.float32)
    h2 = jnp.maximum(h2 + b2_ref[...], 0.0)
    h3 = jnp.sum(h2 * w3_ref[...], axis=1, keepdims=True)
    o_ref[...] = h3 + b3_ref[0, 0]


def _mlp(pooled_a, pooled_b, lens, W1a, W1b, b1, W2, b2, W3, b3):
    BB = 512
    grid = (B // BB,)
    return pl.pallas_call(
        _mlp_body,
        grid=grid,
        in_specs=[
            pl.BlockSpec((BB, DA), lambda i: (i, 0)),
            pl.BlockSpec((BB, DT), lambda i: (i, 0)),
            pl.BlockSpec((BB, 1), lambda i: (i, 0)),
            pl.BlockSpec((150, DA), lambda i: (0, 0)),
            pl.BlockSpec((150, DT), lambda i: (0, 0)),
            pl.BlockSpec((1, 150), lambda i: (0, 0)),
            pl.BlockSpec((150, 150), lambda i: (0, 0)),
            pl.BlockSpec((1, 150), lambda i: (0, 0)),
            pl.BlockSpec((1, 150), lambda i: (0, 0)),
            pl.BlockSpec(memory_space=pltpu.MemorySpace.SMEM),
        ],
        out_specs=pl.BlockSpec((BB, 1), lambda i: (i, 0)),
        out_shape=jax.ShapeDtypeStruct((B, 1), jnp.float32),
    )(pooled_a, pooled_b, lens, W1a, W1b, b1, W2, b2, W3, b3)


def kernel(batch, lens, table, W1, b1, W2, b2, W3, b3):
    batcht = batch.T
    tail_table = table[:, TAIL0:]
    pooled_a = _make_pool(_pool_a_body, DA, DA)(batcht, table)
    tail2, pooled_a = lax.optimization_barrier((tail_table, pooled_a))
    pooled_b = _make_pool(_pool_b_body, 128, DT)(batcht, tail2)
    W1b = jnp.pad(W1[:, DA:], ((0, 0), (4, 0)))
    W1a = W1[:, :DA]
    lens2 = lens.reshape(B, 1)
    out = _mlp(pooled_a, pooled_b, lens2, W1a, W1b, b1.reshape(1, 150), W2,
               b2.reshape(1, 150), W3, b3.reshape(1, 1))
    return out.reshape((B,))
